# Initial kernel scaffold; baseline (speedup 1.0000x reference)
import jax
import jax.numpy as jnp
from jax.experimental import pallas as pl


def kernel(feats, edge_index, W1, b1, a1, W2, b2, Wd):
    raise NotImplementedError("write your pallas kernel here")


# speedup vs baseline: 6.9790x; 6.9790x over previous
import functools

import jax
import jax.numpy as jnp
from jax import lax
from jax.experimental import pallas as pl
from jax.experimental.pallas import tpu as pltpu
from jax.experimental.pallas import tpu_sc as plsc

N = 10000
E = 320000
D_IN = 128
D_H = 512

NP = 10240
DUMMY = N
NSUB = 16
RS = NP // NSUB
CH = 128
NCH = 158
EC = NCH * CH
EP = NSUB * EC
SW = 8
TB = 256
GRID = NP // TB

_mesh = plsc.VectorSubcoreMesh(core_axis_name="c", subcore_axis_name="s")
_sc_params = pltpu.CompilerParams(use_tc_tiling_on_sc=False)


def _col0(block):
    return block[:, 0:1]


def _onehot_row(width):
    return (lax.broadcasted_iota(jnp.int32, (1, width), 1) == 0).astype(jnp.float32)


@functools.partial(
    pl.kernel,
    out_type=(
        jax.ShapeDtypeStruct((NP, SW), jnp.float32),
        jax.ShapeDtypeStruct((NP, SW), jnp.float32),
        jax.ShapeDtypeStruct((NP, D_IN), jnp.float32),
    ),
    mesh=_mesh,
    compiler_params=_sc_params,
    scratch_types=[
        pltpu.VMEM((CH,), jnp.int32),
        pltpu.VMEM((CH,), jnp.int32),
        pltpu.VMEM((CH, SW), jnp.float32),
        pltpu.VMEM((CH, D_IN), jnp.float32),
        pltpu.VMEM_SHARED((NP, SW), jnp.float32),
        pltpu.VMEM_SHARED((NP, SW), jnp.float32),
    ],
)
def _k1(src_ref, dst_ref, x_ref, perm_ref, ones_ref, z8_ref,
        dego_ref, degi_ref, xpg_ref,
        sidx, didx, onesb, rowb, hist_s, hist_d):
    c = lax.axis_index("c")
    s = lax.axis_index("s")
    base_r = s * RS

    @pl.when(c == 0)
    def _():
        @pl.loop(0, RS // CH)
        def _(j):
            base = base_r + j * CH
            pltpu.sync_copy(perm_ref.at[pl.ds(base, CH)], sidx)
            pltpu.sync_copy(x_ref.at[sidx], rowb)
            pltpu.sync_copy(rowb, xpg_ref.at[pl.ds(base, CH)])

    @pl.when(c == 1)
    def _():
        pltpu.sync_copy(ones_ref, onesb)
        pltpu.sync_copy(z8_ref.at[pl.ds(base_r, RS)], hist_s.at[pl.ds(base_r, RS)])
        pltpu.sync_copy(z8_ref.at[pl.ds(base_r, RS)], hist_d.at[pl.ds(base_r, RS)])
        plsc.subcore_barrier()

        @pl.loop(0, NCH)
        def _(j):
            base = s * EC + j * CH
            pltpu.sync_copy(src_ref.at[pl.ds(base, CH)], sidx)
            pltpu.sync_copy(onesb, hist_s.at[sidx], add=True)
            pltpu.sync_copy(dst_ref.at[pl.ds(base, CH)], didx)
            pltpu.sync_copy(onesb, hist_d.at[didx], add=True)

        plsc.subcore_barrier()
        pltpu.sync_copy(hist_s.at[pl.ds(base_r, RS)], dego_ref.at[pl.ds(base_r, RS)])
        pltpu.sync_copy(hist_d.at[pl.ds(base_r, RS)], degi_ref.at[pl.ds(base_r, RS)])


@functools.partial(
    pl.kernel,
    out_type=(
        jax.ShapeDtypeStruct((NP, D_IN), jnp.float32),
        jax.ShapeDtypeStruct((NP, D_IN), jnp.float32),
        jax.ShapeDtypeStruct((NP, SW), jnp.float32),
    ),
    mesh=_mesh,
    compiler_params=_sc_params,
    scratch_types=[
        pltpu.VMEM((CH,), jnp.int32),
        pltpu.VMEM((CH,), jnp.int32),
        pltpu.VMEM((CH, D_IN), jnp.float32),
        pltpu.VMEM((CH, SW), jnp.float32),
        pltpu.VMEM_SHARED((NP, D_IN), jnp.float32),
        pltpu.VMEM_SHARED((NP, SW), jnp.float32),
    ],
)
def _k3(src_ref, dst_ref, xa_ref, xb_ref, inorm8_ref, z128_ref, z8_ref,
        yp_ref, yn_ref, g_ref,
        sidx, didx, rowb, nb, yacc, ghist):
    c = lax.axis_index("c")
    s = lax.axis_index("s")
    base_r = s * RS

    pltpu.sync_copy(z128_ref.at[pl.ds(base_r, RS)], yacc.at[pl.ds(base_r, RS)])

    @pl.when(c == 0)
    def _():
        pltpu.sync_copy(z8_ref.at[pl.ds(base_r, RS)], ghist.at[pl.ds(base_r, RS)])

    plsc.subcore_barrier()

    @pl.loop(0, NCH)
    def _(j):
        base = s * EC + j * CH
        pltpu.sync_copy(src_ref.at[pl.ds(base, CH)], sidx)
        pltpu.sync_copy(dst_ref.at[pl.ds(base, CH)], didx)

        @pl.when(c == 0)
        def _():
            pltpu.sync_copy(xa_ref.at[sidx], rowb)
            pltpu.sync_copy(rowb, yacc.at[didx], add=True)
            pltpu.sync_copy(inorm8_ref.at[didx], nb)
            pltpu.sync_copy(nb, ghist.at[sidx], add=True)

        @pl.when(c == 1)
        def _():
            pltpu.sync_copy(xb_ref.at[sidx], rowb)
            pltpu.sync_copy(rowb, yacc.at[didx], add=True)

    plsc.subcore_barrier()

    @pl.when(c == 0)
    def _():
        pltpu.sync_copy(yacc.at[pl.ds(base_r, RS)], yp_ref.at[pl.ds(base_r, RS)])
        pltpu.sync_copy(ghist.at[pl.ds(base_r, RS)], g_ref.at[pl.ds(base_r, RS)])

    @pl.when(c == 1)
    def _():
        pltpu.sync_copy(yacc.at[pl.ds(base_r, RS)], yn_ref.at[pl.ds(base_r, RS)])


@functools.partial(
    pl.kernel,
    out_type=(
        jax.ShapeDtypeStruct((NP, SW), jnp.float32),
        jax.ShapeDtypeStruct((NP, SW), jnp.float32),
    ),
    mesh=_mesh,
    compiler_params=_sc_params,
    scratch_types=[
        pltpu.VMEM((CH,), jnp.int32),
        pltpu.VMEM((CH,), jnp.int32),
        pltpu.VMEM((CH, SW), jnp.float32),
        pltpu.VMEM_SHARED((NP, SW), jnp.float32),
    ],
)
def _k6(src_ref, dst_ref, tp_ref, tn_ref, z8_ref,
        qp_ref, qn_ref,
        sidx, didx, vb, qacc):
    c = lax.axis_index("c")
    s = lax.axis_index("s")
    base_r = s * RS

    pltpu.sync_copy(z8_ref.at[pl.ds(base_r, RS)], qacc.at[pl.ds(base_r, RS)])
    plsc.subcore_barrier()

    @pl.loop(0, NCH)
    def _(j):
        base = s * EC + j * CH
        pltpu.sync_copy(src_ref.at[pl.ds(base, CH)], sidx)
        pltpu.sync_copy(dst_ref.at[pl.ds(base, CH)], didx)

        @pl.when(c == 0)
        def _():
            pltpu.sync_copy(tp_ref.at[sidx], vb)
            pltpu.sync_copy(vb, qacc.at[didx], add=True)

        @pl.when(c == 1)
        def _():
            pltpu.sync_copy(tn_ref.at[sidx], vb)
            pltpu.sync_copy(vb, qacc.at[didx], add=True)

    plsc.subcore_barrier()

    @pl.when(c == 0)
    def _():
        pltpu.sync_copy(qacc.at[pl.ds(base_r, RS)], qp_ref.at[pl.ds(base_r, RS)])

    @pl.when(c == 1)
    def _():
        pltpu.sync_copy(qacc.at[pl.ds(base_r, RS)], qn_ref.at[pl.ds(base_r, RS)])


def _k2_body(dego_ref, degi_ref, x_ref, xpg_ref,
             onorm_ref, inorm_ref, inorm8_ref, xa_ref, xb_ref):
    on = lax.rsqrt(jnp.maximum(_col0(dego_ref[...]), 1.0))
    inn = lax.rsqrt(jnp.maximum(_col0(degi_ref[...]), 1.0))
    onorm_ref[...] = on
    inorm_ref[...] = inn
    inorm8_ref[...] = inn * _onehot_row(SW)
    xa_ref[...] = x_ref[...] * on
    xb_ref[...] = xpg_ref[...] * on


def _k2(dego, degi, x, xpg):
    return pl.pallas_call(
        _k2_body,
        grid=(GRID,),
        in_specs=[
            pl.BlockSpec((TB, SW), lambda i: (i, 0)),
            pl.BlockSpec((TB, SW), lambda i: (i, 0)),
            pl.BlockSpec((TB, D_IN), lambda i: (i, 0)),
            pl.BlockSpec((TB, D_IN), lambda i: (i, 0)),
        ],
        out_specs=[
            pl.BlockSpec((TB, 1), lambda i: (i, 0)),
            pl.BlockSpec((TB, 1), lambda i: (i, 0)),
            pl.BlockSpec((TB, SW), lambda i: (i, 0)),
            pl.BlockSpec((TB, D_IN), lambda i: (i, 0)),
            pl.BlockSpec((TB, D_IN), lambda i: (i, 0)),
        ],
        out_shape=[
            jax.ShapeDtypeStruct((NP, 1), jnp.float32),
            jax.ShapeDtypeStruct((NP, 1), jnp.float32),
            jax.ShapeDtypeStruct((NP, SW), jnp.float32),
            jax.ShapeDtypeStruct((NP, D_IN), jnp.float32),
            jax.ShapeDtypeStruct((NP, D_IN), jnp.float32),
        ],
    )(dego, degi, x, xpg)


def _k4a_body(yp_ref, yn_ref, inorm_ref, onorm_ref, g_ref, w1_ref, b1_ref,
              a1_ref, h1p_ref, h1n_ref, r_ref):
    inn = inorm_ref[...]
    w1 = w1_ref[...]
    b1 = b1_ref[...]
    a1 = a1_ref[...]
    zp = jnp.dot(yp_ref[...] * inn, w1, preferred_element_type=jnp.float32) + b1
    hp = jnp.where(zp >= 0.0, zp, a1 * zp)
    h1p_ref[...] = hp
    zn = jnp.dot(yn_ref[...] * inn, w1, preferred_element_type=jnp.float32) + b1
    h1n_ref[...] = jnp.where(zn >= 0.0, zn, a1 * zn)

    i = pl.program_id(0)
    rowid = i * TB + lax.broadcasted_iota(jnp.int32, (TB, 1), 0)
    gcol = jnp.where(rowid < N, onorm_ref[...] * _col0(g_ref[...]), 0.0)

    @pl.when(i == 0)
    def _():
        r_ref[...] = jnp.zeros_like(r_ref)

    r_ref[...] += jnp.sum(gcol * hp, axis=0, keepdims=True)


def _k4a(yp, yn, inorm, onorm, g, w1, b1, a1):
    return pl.pallas_call(
        _k4a_body,
        grid=(GRID,),
        in_specs=[
            pl.BlockSpec((TB, D_IN), lambda i: (i, 0)),
            pl.BlockSpec((TB, D_IN), lambda i: (i, 0)),
            pl.BlockSpec((TB, 1), lambda i: (i, 0)),
            pl.BlockSpec((TB, 1), lambda i: (i, 0)),
            pl.BlockSpec((TB, SW), lambda i: (i, 0)),
            pl.BlockSpec((D_IN, D_H), lambda i: (0, 0)),
            pl.BlockSpec((1, D_H), lambda i: (0, 0)),
            pl.BlockSpec((1, D_H), lambda i: (0, 0)),
        ],
        out_specs=[
            pl.BlockSpec((TB, D_H), lambda i: (i, 0)),
            pl.BlockSpec((TB, D_H), lambda i: (i, 0)),
            pl.BlockSpec((1, D_H), lambda i: (0, 0)),
        ],
        out_shape=[
            jax.ShapeDtypeStruct((NP, D_H), jnp.float32),
            jax.ShapeDtypeStruct((NP, D_H), jnp.float32),
            jax.ShapeDtypeStruct((1, D_H), jnp.float32),
        ],
    )(yp, yn, inorm, onorm, g, w1, b1, a1)


def _k4b_body(r_ref, w2_ref, w2t_ref, wdt_ref, b2_ref, wt_ref, beta_ref):
    m = jnp.dot(r_ref[...], w2_ref[...], preferred_element_type=jnp.float32) / N
    m = m + b2_ref[...]
    sgm = 1.0 / (1.0 + jnp.exp(-m))
    ws = jnp.dot(sgm, wdt_ref[...], preferred_element_type=jnp.float32)
    wt_ref[...] = jnp.dot(ws, w2t_ref[...], preferred_element_type=jnp.float32)
    beta_ref[...] = jnp.sum(b2_ref[...] * ws, axis=1, keepdims=True)


def _k4b(r, w2, w2t, wdt, b2):
    return pl.pallas_call(
        _k4b_body,
        out_shape=[
            jax.ShapeDtypeStruct((1, D_H), jnp.float32),
            jax.ShapeDtypeStruct((1, 1), jnp.float32),
        ],
    )(r, w2, w2t, wdt, b2)


def _k4c_body(h1p_ref, h1n_ref, onorm_ref, wt_ref, tp_ref, tn_ref):
    wt = wt_ref[...]
    on = onorm_ref[...]
    oh = _onehot_row(SW)
    tp_ref[...] = on * jnp.sum(h1p_ref[...] * wt, axis=1, keepdims=True) * oh
    tn_ref[...] = on * jnp.sum(h1n_ref[...] * wt, axis=1, keepdims=True) * oh


def _k4c(h1p, h1n, onorm, wt):
    return pl.pallas_call(
        _k4c_body,
        grid=(GRID,),
        in_specs=[
            pl.BlockSpec((TB, D_H), lambda i: (i, 0)),
            pl.BlockSpec((TB, D_H), lambda i: (i, 0)),
            pl.BlockSpec((TB, 1), lambda i: (i, 0)),
            pl.BlockSpec((1, D_H), lambda i: (0, 0)),
        ],
        out_specs=[
            pl.BlockSpec((TB, SW), lambda i: (i, 0)),
            pl.BlockSpec((TB, SW), lambda i: (i, 0)),
        ],
        out_shape=[
            jax.ShapeDtypeStruct((NP, SW), jnp.float32),
            jax.ShapeDtypeStruct((NP, SW), jnp.float32),
        ],
    )(h1p, h1n, onorm, wt)


def _k5_body(qp_ref, qn_ref, inorm_ref, beta_ref, loss_ref):
    rows = NP // 128
    rowid = lax.broadcasted_iota(jnp.int32, (rows, 128), 0) * 128 + \
        lax.broadcasted_iota(jnp.int32, (rows, 128), 1)
    mask = rowid < N
    inn = inorm_ref[...]
    beta = beta_ref[...]

    sp = inn * qp_ref[...] + beta
    l1 = jnp.maximum(sp, 0.0) - sp + jnp.log(1.0 + jnp.exp(-jnp.abs(sp)))
    l1 = jnp.where(mask, l1, 0.0)

    sn = inn * qn_ref[...] + beta
    l2 = jnp.maximum(sn, 0.0) + jnp.log(1.0 + jnp.exp(-jnp.abs(sn)))
    l2 = jnp.where(mask, l2, 0.0)

    total = jnp.sum(l1 + l2, axis=0, keepdims=True)
    loss_ref[...] = jnp.sum(total, axis=1, keepdims=True) / N


def _k5(qp2, qn2, inorm2, beta):
    return pl.pallas_call(
        _k5_body,
        out_shape=jax.ShapeDtypeStruct((1, 1), jnp.float32),
    )(qp2, qn2, inorm2, beta)


def kernel(feats, edge_index, W1, b1, a1, W2, b2, Wd):
    src = edge_index[0].astype(jnp.int32)
    dst = edge_index[1].astype(jnp.int32)
    pad_e = jnp.full((EP - E,), DUMMY, dtype=jnp.int32)
    src_p = jnp.concatenate([src, pad_e])
    dst_p = jnp.concatenate([dst, pad_e])

    x_p = jnp.pad(feats.astype(jnp.float32), ((0, NP - N), (0, 0)))
    perm = jax.random.permutation(jax.random.key(1), N).astype(jnp.int32)
    perm_p = jnp.concatenate([perm, jnp.full((NP - N,), DUMMY, dtype=jnp.int32)])

    ones8 = jnp.zeros((CH, SW), jnp.float32).at[:, 0].set(1.0)
    z8 = jnp.zeros((NP, SW), jnp.float32)
    z128 = jnp.zeros((NP, D_IN), jnp.float32)

    dego, degi, xpg = _k1(src_p, dst_p, x_p, perm_p, ones8, z8)
    onorm, inorm, inorm8, xa, xb = _k2(dego, degi, x_p, xpg)
    yp, yn, g = _k3(src_p, dst_p, xa, xb, inorm8, z128, z8)

    w1 = W1.astype(jnp.float32)
    b1r = b1.astype(jnp.float32).reshape(1, D_H)
    a1r = a1.astype(jnp.float32).reshape(1, D_H)
    h1p, h1n, r = _k4a(yp, yn, inorm, onorm, g, w1, b1r, a1r)

    w2 = W2.astype(jnp.float32)
    b2r = b2.astype(jnp.float32).reshape(1, D_H)
    wt, beta = _k4b(r, w2, w2.T, Wd.astype(jnp.float32).T, b2r)

    tp, tn = _k4c(h1p, h1n, onorm, wt)
    qp, qn = _k6(src_p, dst_p, tp, tn, z8)

    loss = _k5(qp[:, 0].reshape(NP // 128, 128), qn[:, 0].reshape(NP // 128, 128),
               inorm.reshape(NP // 128, 128), beta)
    return loss[0, 0]


# speedup vs baseline: 10.1016x; 1.4474x over previous
import functools

import jax
import jax.numpy as jnp
from jax import lax
from jax.experimental import pallas as pl
from jax.experimental.pallas import tpu as pltpu
from jax.experimental.pallas import tpu_sc as plsc

N = 10000
E = 320000
D_IN = 128
D_H = 512

NP = 10240
DUMMY = N
NSUB = 16
RS = NP // NSUB
CH = 128
NCH = 160
EC = NCH * CH
EP = NSUB * EC
UNR = 8
NI = NCH // UNR
SW = 8
TB = 256
GRID = NP // TB

_mesh = plsc.VectorSubcoreMesh(core_axis_name="c", subcore_axis_name="s")
_sc_params = pltpu.CompilerParams(use_tc_tiling_on_sc=False)


def _col0(block):
    return block[:, 0:1]


def _issue_idx(arr_ref, buf, sem, base):
    pltpu.async_copy(arr_ref.at[pl.ds(base, CH)], buf, sem)


def _wait_bytes(src, dst, sem):
    pltpu.make_async_copy(src, dst, sem).wait()


def _onehot_row(width):
    return (lax.broadcasted_iota(jnp.int32, (1, width), 1) == 0).astype(jnp.float32)


@functools.partial(
    pl.kernel,
    out_type=(
        jax.ShapeDtypeStruct((NP, SW), jnp.float32),
        jax.ShapeDtypeStruct((NP, SW), jnp.float32),
        jax.ShapeDtypeStruct((NP, D_IN), jnp.float32),
    ),
    mesh=_mesh,
    compiler_params=_sc_params,
    scratch_types=(
        [pltpu.VMEM((CH,), jnp.int32)] * UNR +
        [pltpu.VMEM((CH,), jnp.int32)] +
        [pltpu.VMEM((CH, SW), jnp.float32)] +
        [pltpu.VMEM((CH, D_IN), jnp.float32)] +
        [pltpu.VMEM_SHARED((NP, SW), jnp.float32)] +
        [pltpu.SemaphoreType.DMA] * UNR +
        [pltpu.SemaphoreType.DMA] * 4
    ),
)
def _k1(src_ref, dst_ref, x_ref, perm_ref, ones_ref, z8_ref,
        dego_ref, degi_ref, xpg_ref, *scr):
    idxb = scr[0:UNR]
    pidx = scr[UNR]
    onesb = scr[UNR + 1]
    rowb = scr[UNR + 2]
    hist = scr[UNR + 3]
    isem = scr[UNR + 4:UNR + 4 + UNR]
    ssem = scr[UNR + 4 + UNR:UNR + 8 + UNR]

    c = lax.axis_index("c")
    s = lax.axis_index("s")
    base_r = s * RS

    def hist_pipeline(earr, out_ref):
        pltpu.sync_copy(ones_ref, onesb)
        pltpu.sync_copy(z8_ref.at[pl.ds(base_r, RS)], hist.at[pl.ds(base_r, RS)])
        plsc.subcore_barrier()

        for k in range(4):
            _issue_idx(earr, idxb[k], isem[k], s * EC + k * CH)

        @pl.loop(0, NI)
        def _(o):
            for u in range(UNR):
                x, r = u, u % 4

                def waits():
                    _wait_bytes(onesb, hist.at[idxb[x]], ssem[r])

                if u >= 4:
                    waits()
                else:
                    pl.when(o > 0)(waits)

                _wait_bytes(earr.at[pl.ds(0, CH)], idxb[x], isem[x])
                pltpu.async_copy(onesb, hist.at[idxb[x]], ssem[r], add=True)

                x4 = (u + 4) % UNR
                base4 = s * EC + (o * UNR + u + 4) * CH

                def prefetch():
                    _issue_idx(earr, idxb[x4], isem[x4], base4)

                if u < 4:
                    prefetch()
                else:
                    pl.when(o < NI - 1)(prefetch)

        for k in range(4):
            _wait_bytes(onesb, hist.at[idxb[k]], ssem[k])

        plsc.subcore_barrier()
        pltpu.sync_copy(hist.at[pl.ds(base_r, RS)], out_ref.at[pl.ds(base_r, RS)])

    @pl.when(c == 0)
    def _():
        @pl.loop(0, RS // CH)
        def _(j):
            base = base_r + j * CH
            pltpu.sync_copy(perm_ref.at[pl.ds(base, CH)], pidx)
            pltpu.sync_copy(x_ref.at[pidx], rowb)
            pltpu.sync_copy(rowb, xpg_ref.at[pl.ds(base, CH)])

        hist_pipeline(src_ref, dego_ref)

    @pl.when(c == 1)
    def _():
        hist_pipeline(dst_ref, degi_ref)


@functools.partial(
    pl.kernel,
    out_type=(
        jax.ShapeDtypeStruct((NP, D_IN), jnp.float32),
        jax.ShapeDtypeStruct((NP, D_IN), jnp.float32),
        jax.ShapeDtypeStruct((NP, SW), jnp.float32),
    ),
    mesh=_mesh,
    compiler_params=_sc_params,
    scratch_types=(
        [pltpu.VMEM((CH,), jnp.int32)] * (2 * UNR) +
        [pltpu.VMEM((CH, D_IN), jnp.float32)] * 2 +
        [pltpu.VMEM((CH, SW), jnp.float32)] * 2 +
        [pltpu.VMEM_SHARED((NP, D_IN), jnp.float32)] +
        [pltpu.VMEM_SHARED((NP, SW), jnp.float32)] +
        [pltpu.SemaphoreType.DMA] * (UNR + 8)
    ),
)
def _k3(src_ref, dst_ref, xa_ref, xb_ref, inorm8_ref, z128_ref, z8_ref,
        yp_ref, yn_ref, g_ref, *scr):
    sidx = scr[0:UNR]
    didx = scr[UNR:2 * UNR]
    rowb = scr[2 * UNR:2 * UNR + 2]
    nb = scr[2 * UNR + 2:2 * UNR + 4]
    yacc = scr[2 * UNR + 4]
    ghist = scr[2 * UNR + 5]
    sems = scr[2 * UNR + 6:]
    isem = sems[0:UNR]
    gsem = sems[UNR:UNR + 2]
    ssem = sems[UNR + 2:UNR + 4]
    nsem = sems[UNR + 4:UNR + 6]
    qsem = sems[UNR + 6:UNR + 8]

    c = lax.axis_index("c")
    s = lax.axis_index("s")
    base_r = s * RS

    def agg_pipeline(tab_ref, out_ref, with_g):
        pltpu.sync_copy(z128_ref.at[pl.ds(base_r, RS)], yacc.at[pl.ds(base_r, RS)])
        if with_g:
            pltpu.sync_copy(z8_ref.at[pl.ds(base_r, RS)], ghist.at[pl.ds(base_r, RS)])
        plsc.subcore_barrier()

        for k in range(4):
            _issue_idx(src_ref, sidx[k], isem[k], s * EC + k * CH)
            _issue_idx(dst_ref, didx[k], isem[k], s * EC + k * CH)

        @pl.loop(0, NI)
        def _(o):
            for u in range(UNR):
                x, r = u, u % 2

                def retire():
                    _wait_bytes(rowb[r], yacc.at[didx[x]], ssem[r])
                    if with_g:
                        _wait_bytes(nb[r], ghist.at[sidx[x]], qsem[r])

                if u >= 2:
                    retire()
                else:
                    pl.when(o > 0)(retire)

                _wait_bytes(src_ref.at[pl.ds(0, CH)], sidx[x], isem[x])
                _wait_bytes(dst_ref.at[pl.ds(0, CH)], didx[x], isem[x])
                pltpu.async_copy(tab_ref.at[sidx[x]], rowb[r], gsem[r])
                if with_g:
                    pltpu.async_copy(inorm8_ref.at[didx[x]], nb[r], nsem[r])

                x4 = (u + 4) % UNR
                base4 = s * EC + (o * UNR + u + 4) * CH

                def prefetch():
                    _issue_idx(src_ref, sidx[x4], isem[x4], base4)
                    _issue_idx(dst_ref, didx[x4], isem[x4], base4)

                if u < 4:
                    prefetch()
                else:
                    pl.when(o < NI - 1)(prefetch)

                x1, r1 = (u - 1) % UNR, (u - 1) % 2

                def scatter1():
                    _wait_bytes(tab_ref.at[sidx[x1]], rowb[r1], gsem[r1])
                    pltpu.async_copy(rowb[r1], yacc.at[didx[x1]], ssem[r1], add=True)
                    if with_g:
                        _wait_bytes(inorm8_ref.at[didx[x1]], nb[r1], nsem[r1])
                        pltpu.async_copy(nb[r1], ghist.at[sidx[x1]], qsem[r1], add=True)

                if u >= 1:
                    scatter1()
                else:
                    pl.when(o > 0)(scatter1)

        ctail = NCH - 1
        x1, r1 = ctail % UNR, ctail % 2
        _wait_bytes(tab_ref.at[sidx[x1]], rowb[r1], gsem[r1])
        pltpu.async_copy(rowb[r1], yacc.at[didx[x1]], ssem[r1], add=True)
        if with_g:
            _wait_bytes(inorm8_ref.at[didx[x1]], nb[r1], nsem[r1])
            pltpu.async_copy(nb[r1], ghist.at[sidx[x1]], qsem[r1], add=True)

        for k in range(2):
            _wait_bytes(rowb[k], yacc.at[didx[k]], ssem[k])
            if with_g:
                _wait_bytes(nb[k], ghist.at[sidx[k]], qsem[k])

        plsc.subcore_barrier()
        pltpu.sync_copy(yacc.at[pl.ds(base_r, RS)], out_ref.at[pl.ds(base_r, RS)])
        if with_g:
            pltpu.sync_copy(ghist.at[pl.ds(base_r, RS)], g_ref.at[pl.ds(base_r, RS)])

    @pl.when(c == 0)
    def _():
        agg_pipeline(xa_ref, yp_ref, True)

    @pl.when(c == 1)
    def _():
        agg_pipeline(xb_ref, yn_ref, False)


@functools.partial(
    pl.kernel,
    out_type=(
        jax.ShapeDtypeStruct((NP, SW), jnp.float32),
        jax.ShapeDtypeStruct((NP, SW), jnp.float32),
    ),
    mesh=_mesh,
    compiler_params=_sc_params,
    scratch_types=(
        [pltpu.VMEM((CH,), jnp.int32)] * (2 * UNR) +
        [pltpu.VMEM((CH, SW), jnp.float32)] * 4 +
        [pltpu.VMEM_SHARED((NP, SW), jnp.float32)] +
        [pltpu.SemaphoreType.DMA] * (UNR + 8)
    ),
)
def _k6(src_ref, dst_ref, tp_ref, tn_ref, z8_ref,
        qp_ref, qn_ref, *scr):
    sidx = scr[0:UNR]
    didx = scr[UNR:2 * UNR]
    vb = scr[2 * UNR:2 * UNR + 4]
    qacc = scr[2 * UNR + 4]
    sems = scr[2 * UNR + 5:]
    isem = sems[0:UNR]
    gsem = sems[UNR:UNR + 4]
    ssem = sems[UNR + 4:UNR + 8]

    c = lax.axis_index("c")
    s = lax.axis_index("s")
    base_r = s * RS

    def seg_pipeline(tab_ref, out_ref):
        pltpu.sync_copy(z8_ref.at[pl.ds(base_r, RS)], qacc.at[pl.ds(base_r, RS)])
        plsc.subcore_barrier()

        for k in range(4):
            _issue_idx(src_ref, sidx[k], isem[k], s * EC + k * CH)
            _issue_idx(dst_ref, didx[k], isem[k], s * EC + k * CH)

        @pl.loop(0, NI)
        def _(o):
            for u in range(UNR):
                x, r = u, u % 4

                def retire():
                    _wait_bytes(vb[r], qacc.at[didx[x]], ssem[r])

                if u >= 4:
                    retire()
                else:
                    pl.when(o > 0)(retire)

                _wait_bytes(src_ref.at[pl.ds(0, CH)], sidx[x], isem[x])
                _wait_bytes(dst_ref.at[pl.ds(0, CH)], didx[x], isem[x])
                pltpu.async_copy(tab_ref.at[sidx[x]], vb[r], gsem[r])

                x4 = (u + 4) % UNR
                base4 = s * EC + (o * UNR + u + 4) * CH

                def prefetch():
                    _issue_idx(src_ref, sidx[x4], isem[x4], base4)
                    _issue_idx(dst_ref, didx[x4], isem[x4], base4)

                if u < 4:
                    prefetch()
                else:
                    pl.when(o < NI - 1)(prefetch)

                x2, r2 = (u - 2) % UNR, (u - 2) % 4

                def scatter2():
                    _wait_bytes(tab_ref.at[sidx[x2]], vb[r2], gsem[r2])
                    pltpu.async_copy(vb[r2], qacc.at[didx[x2]], ssem[r2], add=True)

                if u >= 2:
                    scatter2()
                else:
                    pl.when(o > 0)(scatter2)

        for ctail in (NCH - 2, NCH - 1):
            x2, r2 = ctail % UNR, ctail % 4
            _wait_bytes(tab_ref.at[sidx[x2]], vb[r2], gsem[r2])
            pltpu.async_copy(vb[r2], qacc.at[didx[x2]], ssem[r2], add=True)

        for k in range(4):
            _wait_bytes(vb[k], qacc.at[didx[k]], ssem[k])

        plsc.subcore_barrier()
        pltpu.sync_copy(qacc.at[pl.ds(base_r, RS)], out_ref.at[pl.ds(base_r, RS)])

    @pl.when(c == 0)
    def _():
        seg_pipeline(tp_ref, qp_ref)

    @pl.when(c == 1)
    def _():
        seg_pipeline(tn_ref, qn_ref)


def _k2_body(dego_ref, degi_ref, x_ref, xpg_ref,
             onorm_ref, inorm_ref, inorm8_ref, xa_ref, xb_ref):
    on = lax.rsqrt(jnp.maximum(_col0(dego_ref[...]), 1.0))
    inn = lax.rsqrt(jnp.maximum(_col0(degi_ref[...]), 1.0))
    onorm_ref[...] = on
    inorm_ref[...] = inn
    inorm8_ref[...] = inn * _onehot_row(SW)
    xa_ref[...] = x_ref[...] * on
    xb_ref[...] = xpg_ref[...] * on


def _k2(dego, degi, x, xpg):
    return pl.pallas_call(
        _k2_body,
        grid=(GRID,),
        in_specs=[
            pl.BlockSpec((TB, SW), lambda i: (i, 0)),
            pl.BlockSpec((TB, SW), lambda i: (i, 0)),
            pl.BlockSpec((TB, D_IN), lambda i: (i, 0)),
            pl.BlockSpec((TB, D_IN), lambda i: (i, 0)),
        ],
        out_specs=[
            pl.BlockSpec((TB, 1), lambda i: (i, 0)),
            pl.BlockSpec((TB, 1), lambda i: (i, 0)),
            pl.BlockSpec((TB, SW), lambda i: (i, 0)),
            pl.BlockSpec((TB, D_IN), lambda i: (i, 0)),
            pl.BlockSpec((TB, D_IN), lambda i: (i, 0)),
        ],
        out_shape=[
            jax.ShapeDtypeStruct((NP, 1), jnp.float32),
            jax.ShapeDtypeStruct((NP, 1), jnp.float32),
            jax.ShapeDtypeStruct((NP, SW), jnp.float32),
            jax.ShapeDtypeStruct((NP, D_IN), jnp.float32),
            jax.ShapeDtypeStruct((NP, D_IN), jnp.float32),
        ],
    )(dego, degi, x, xpg)


def _k4a_body(yp_ref, yn_ref, inorm_ref, onorm_ref, g_ref, w1_ref, b1_ref,
              a1_ref, h1p_ref, h1n_ref, r_ref):
    inn = inorm_ref[...]
    w1 = w1_ref[...]
    b1 = b1_ref[...]
    a1 = a1_ref[...]
    zp = jnp.dot(yp_ref[...] * inn, w1, preferred_element_type=jnp.float32) + b1
    hp = jnp.where(zp >= 0.0, zp, a1 * zp)
    h1p_ref[...] = hp
    zn = jnp.dot(yn_ref[...] * inn, w1, preferred_element_type=jnp.float32) + b1
    h1n_ref[...] = jnp.where(zn >= 0.0, zn, a1 * zn)

    i = pl.program_id(0)
    rowid = i * TB + lax.broadcasted_iota(jnp.int32, (TB, 1), 0)
    gcol = jnp.where(rowid < N, onorm_ref[...] * _col0(g_ref[...]), 0.0)

    @pl.when(i == 0)
    def _():
        r_ref[...] = jnp.zeros_like(r_ref)

    r_ref[...] += jnp.sum(gcol * hp, axis=0, keepdims=True)


def _k4a(yp, yn, inorm, onorm, g, w1, b1, a1):
    return pl.pallas_call(
        _k4a_body,
        grid=(GRID,),
        in_specs=[
            pl.BlockSpec((TB, D_IN), lambda i: (i, 0)),
            pl.BlockSpec((TB, D_IN), lambda i: (i, 0)),
            pl.BlockSpec((TB, 1), lambda i: (i, 0)),
            pl.BlockSpec((TB, 1), lambda i: (i, 0)),
            pl.BlockSpec((TB, SW), lambda i: (i, 0)),
            pl.BlockSpec((D_IN, D_H), lambda i: (0, 0)),
            pl.BlockSpec((1, D_H), lambda i: (0, 0)),
            pl.BlockSpec((1, D_H), lambda i: (0, 0)),
        ],
        out_specs=[
            pl.BlockSpec((TB, D_H), lambda i: (i, 0)),
            pl.BlockSpec((TB, D_H), lambda i: (i, 0)),
            pl.BlockSpec((1, D_H), lambda i: (0, 0)),
        ],
        out_shape=[
            jax.ShapeDtypeStruct((NP, D_H), jnp.float32),
            jax.ShapeDtypeStruct((NP, D_H), jnp.float32),
            jax.ShapeDtypeStruct((1, D_H), jnp.float32),
        ],
    )(yp, yn, inorm, onorm, g, w1, b1, a1)


def _k4b_body(r_ref, w2_ref, w2t_ref, wdt_ref, b2_ref, wt_ref, beta_ref):
    m = jnp.dot(r_ref[...], w2_ref[...], preferred_element_type=jnp.float32) / N
    m = m + b2_ref[...]
    sgm = 1.0 / (1.0 + jnp.exp(-m))
    ws = jnp.dot(sgm, wdt_ref[...], preferred_element_type=jnp.float32)
    wt_ref[...] = jnp.dot(ws, w2t_ref[...], preferred_element_type=jnp.float32)
    beta_ref[...] = jnp.sum(b2_ref[...] * ws, axis=1, keepdims=True)


def _k4b(r, w2, w2t, wdt, b2):
    return pl.pallas_call(
        _k4b_body,
        out_shape=[
            jax.ShapeDtypeStruct((1, D_H), jnp.float32),
            jax.ShapeDtypeStruct((1, 1), jnp.float32),
        ],
    )(r, w2, w2t, wdt, b2)


def _k4c_body(h1p_ref, h1n_ref, onorm_ref, wt_ref, tp_ref, tn_ref):
    wt = wt_ref[...]
    on = onorm_ref[...]
    oh = _onehot_row(SW)
    tp_ref[...] = on * jnp.sum(h1p_ref[...] * wt, axis=1, keepdims=True) * oh
    tn_ref[...] = on * jnp.sum(h1n_ref[...] * wt, axis=1, keepdims=True) * oh


def _k4c(h1p, h1n, onorm, wt):
    return pl.pallas_call(
        _k4c_body,
        grid=(GRID,),
        in_specs=[
            pl.BlockSpec((TB, D_H), lambda i: (i, 0)),
            pl.BlockSpec((TB, D_H), lambda i: (i, 0)),
            pl.BlockSpec((TB, 1), lambda i: (i, 0)),
            pl.BlockSpec((1, D_H), lambda i: (0, 0)),
        ],
        out_specs=[
            pl.BlockSpec((TB, SW), lambda i: (i, 0)),
            pl.BlockSpec((TB, SW), lambda i: (i, 0)),
        ],
        out_shape=[
            jax.ShapeDtypeStruct((NP, SW), jnp.float32),
            jax.ShapeDtypeStruct((NP, SW), jnp.float32),
        ],
    )(h1p, h1n, onorm, wt)


def _k5_body(qp_ref, qn_ref, inorm_ref, beta_ref, loss_ref):
    rows = NP // 128
    rowid = lax.broadcasted_iota(jnp.int32, (rows, 128), 0) * 128 + \
        lax.broadcasted_iota(jnp.int32, (rows, 128), 1)
    mask = rowid < N
    inn = inorm_ref[...]
    beta = beta_ref[...]

    sp = inn * qp_ref[...] + beta
    l1 = jnp.maximum(sp, 0.0) - sp + jnp.log(1.0 + jnp.exp(-jnp.abs(sp)))
    l1 = jnp.where(mask, l1, 0.0)

    sn = inn * qn_ref[...] + beta
    l2 = jnp.maximum(sn, 0.0) + jnp.log(1.0 + jnp.exp(-jnp.abs(sn)))
    l2 = jnp.where(mask, l2, 0.0)

    total = jnp.sum(l1 + l2, axis=0, keepdims=True)
    loss_ref[...] = jnp.sum(total, axis=1, keepdims=True) / N


def _k5(qp2, qn2, inorm2, beta):
    return pl.pallas_call(
        _k5_body,
        out_shape=jax.ShapeDtypeStruct((1, 1), jnp.float32),
    )(qp2, qn2, inorm2, beta)


def kernel(feats, edge_index, W1, b1, a1, W2, b2, Wd):
    src = edge_index[0].astype(jnp.int32)
    dst = edge_index[1].astype(jnp.int32)
    pad_e = jnp.full((EP - E,), DUMMY, dtype=jnp.int32)
    src_p = jnp.concatenate([src, pad_e])
    dst_p = jnp.concatenate([dst, pad_e])

    x_p = jnp.pad(feats.astype(jnp.float32), ((0, NP - N), (0, 0)))
    perm = jax.random.permutation(jax.random.key(1), N).astype(jnp.int32)
    perm_p = jnp.concatenate([perm, jnp.full((NP - N,), DUMMY, dtype=jnp.int32)])

    ones8 = jnp.zeros((CH, SW), jnp.float32).at[:, 0].set(1.0)
    z8 = jnp.zeros((NP, SW), jnp.float32)
    z128 = jnp.zeros((NP, D_IN), jnp.float32)

    dego, degi, xpg = _k1(src_p, dst_p, x_p, perm_p, ones8, z8)
    onorm, inorm, inorm8, xa, xb = _k2(dego, degi, x_p, xpg)
    yp, yn, g = _k3(src_p, dst_p, xa, xb, inorm8, z128, z8)

    w1 = W1.astype(jnp.float32)
    b1r = b1.astype(jnp.float32).reshape(1, D_H)
    a1r = a1.astype(jnp.float32).reshape(1, D_H)
    h1p, h1n, r = _k4a(yp, yn, inorm, onorm, g, w1, b1r, a1r)

    w2 = W2.astype(jnp.float32)
    b2r = b2.astype(jnp.float32).reshape(1, D_H)
    wt, beta = _k4b(r, w2, w2.T, Wd.astype(jnp.float32).T, b2r)

    tp, tn = _k4c(h1p, h1n, onorm, wt)
    qp, qn = _k6(src_p, dst_p, tp, tn, z8)

    loss = _k5(qp[:, 0].reshape(NP // 128, 128), qn[:, 0].reshape(NP // 128, 128),
               inorm.reshape(NP // 128, 128), beta)
    return loss[0, 0]


# speedup vs baseline: 14.9637x; 1.4813x over previous
import functools

import jax
import jax.numpy as jnp
from jax import lax
from jax.experimental import pallas as pl
from jax.experimental.pallas import tpu as pltpu
from jax.experimental.pallas import tpu_sc as plsc

N = 10000
E = 320000
D_IN = 128
D_H = 512

NP = 10240
DUMMY = N
NSUB = 16
RS = NP // NSUB
CH = 128
NCH = 160
EC = NCH * CH
EP = NSUB * EC
UNR = 8
NI = NCH // UNR
SW = 8
TB = 256
GRID = NP // TB

_mesh = plsc.VectorSubcoreMesh(core_axis_name="c", subcore_axis_name="s")
_sc_params = pltpu.CompilerParams(use_tc_tiling_on_sc=False)


def _col0(block):
    return block[:, 0:1]


def _issue_idx(arr_ref, buf, sem, base):
    pltpu.async_copy(arr_ref.at[pl.ds(base, CH)], buf, sem)


def _wait_bytes(src, dst, sem):
    pltpu.make_async_copy(src, dst, sem).wait()


def _onehot_row(width):
    return (lax.broadcasted_iota(jnp.int32, (1, width), 1) == 0).astype(jnp.float32)


@functools.partial(
    pl.kernel,
    out_type=(
        jax.ShapeDtypeStruct((NP, SW), jnp.float32),
        jax.ShapeDtypeStruct((NP, SW), jnp.float32),
        jax.ShapeDtypeStruct((NP, D_IN), jnp.float32),
    ),
    mesh=_mesh,
    compiler_params=_sc_params,
    scratch_types=(
        [pltpu.VMEM((CH,), jnp.int32)] * UNR +
        [pltpu.VMEM((CH,), jnp.int32)] +
        [pltpu.VMEM((CH, SW), jnp.float32)] +
        [pltpu.VMEM((CH, D_IN), jnp.float32)] +
        [pltpu.VMEM_SHARED((NP, SW), jnp.float32)] +
        [pltpu.SemaphoreType.DMA] * UNR +
        [pltpu.SemaphoreType.DMA] * 4
    ),
)
def _k1(src_ref, dst_ref, x_ref, perm_ref, ones_ref, z8_ref,
        dego_ref, degi_ref, xpg_ref, *scr):
    idxb = scr[0:UNR]
    pidx = scr[UNR]
    onesb = scr[UNR + 1]
    rowb = scr[UNR + 2]
    hist = scr[UNR + 3]
    isem = scr[UNR + 4:UNR + 4 + UNR]
    ssem = scr[UNR + 4 + UNR:UNR + 8 + UNR]

    c = lax.axis_index("c")
    s = lax.axis_index("s")
    base_r = s * RS

    def hist_pipeline(earr, out_ref):
        pltpu.sync_copy(ones_ref, onesb)
        pltpu.sync_copy(z8_ref.at[pl.ds(base_r, RS)], hist.at[pl.ds(base_r, RS)])
        plsc.subcore_barrier()

        for k in range(4):
            _issue_idx(earr, idxb[k], isem[k], s * EC + k * CH)

        @pl.loop(0, NI)
        def _(o):
            for u in range(UNR):
                x, r = u, u % 4

                def waits():
                    _wait_bytes(onesb, hist.at[idxb[x]], ssem[r])

                if u >= 4:
                    waits()
                else:
                    pl.when(o > 0)(waits)

                _wait_bytes(earr.at[pl.ds(0, CH)], idxb[x], isem[x])
                pltpu.async_copy(onesb, hist.at[idxb[x]], ssem[r], add=True)

                x4 = (u + 4) % UNR
                base4 = s * EC + (o * UNR + u + 4) * CH

                def prefetch():
                    _issue_idx(earr, idxb[x4], isem[x4], base4)

                if u < 4:
                    prefetch()
                else:
                    pl.when(o < NI - 1)(prefetch)

        for k in range(4):
            _wait_bytes(onesb, hist.at[idxb[k]], ssem[k])

        plsc.subcore_barrier()
        pltpu.sync_copy(hist.at[pl.ds(base_r, RS)], out_ref.at[pl.ds(base_r, RS)])

    @pl.when(c == 0)
    def _():
        @pl.loop(0, RS // CH)
        def _(j):
            base = base_r + j * CH
            pltpu.sync_copy(perm_ref.at[pl.ds(base, CH)], pidx)
            pltpu.sync_copy(x_ref.at[pidx], rowb)
            pltpu.sync_copy(rowb, xpg_ref.at[pl.ds(base, CH)])

        hist_pipeline(src_ref, dego_ref)

    @pl.when(c == 1)
    def _():
        hist_pipeline(dst_ref, degi_ref)


@functools.partial(
    pl.kernel,
    out_type=(
        jax.ShapeDtypeStruct((NP, D_IN), jnp.bfloat16),
        jax.ShapeDtypeStruct((NP, D_IN), jnp.bfloat16),
        jax.ShapeDtypeStruct((NP, SW), jnp.float32),
    ),
    mesh=_mesh,
    compiler_params=_sc_params,
    scratch_types=(
        [pltpu.VMEM((CH,), jnp.int32)] * (2 * UNR) +
        [pltpu.VMEM((CH, D_IN), jnp.bfloat16)] * 2 +
        [pltpu.VMEM((CH, SW), jnp.float32)] * 2 +
        [pltpu.VMEM_SHARED((NP, D_IN), jnp.bfloat16)] +
        [pltpu.VMEM_SHARED((NP, SW), jnp.float32)] +
        [pltpu.SemaphoreType.DMA] * (UNR + 8)
    ),
)
def _k3(src_ref, dst_ref, xa_ref, xb_ref, inorm8_ref, z128_ref, z8_ref,
        yp_ref, yn_ref, g_ref, *scr):
    sidx = scr[0:UNR]
    didx = scr[UNR:2 * UNR]
    rowb = scr[2 * UNR:2 * UNR + 2]
    nb = scr[2 * UNR + 2:2 * UNR + 4]
    yacc = scr[2 * UNR + 4]
    ghist = scr[2 * UNR + 5]
    sems = scr[2 * UNR + 6:]
    isem = sems[0:UNR]
    gsem = sems[UNR:UNR + 2]
    ssem = sems[UNR + 2:UNR + 4]
    nsem = sems[UNR + 4:UNR + 6]
    qsem = sems[UNR + 6:UNR + 8]

    c = lax.axis_index("c")
    s = lax.axis_index("s")
    base_r = s * RS

    def agg_pipeline(tab_ref, out_ref, with_g):
        pltpu.sync_copy(z128_ref.at[pl.ds(base_r, RS)], yacc.at[pl.ds(base_r, RS)])
        if with_g:
            pltpu.sync_copy(z8_ref.at[pl.ds(base_r, RS)], ghist.at[pl.ds(base_r, RS)])
        plsc.subcore_barrier()

        for k in range(4):
            _issue_idx(src_ref, sidx[k], isem[k], s * EC + k * CH)
            _issue_idx(dst_ref, didx[k], isem[k], s * EC + k * CH)

        @pl.loop(0, NI)
        def _(o):
            for u in range(UNR):
                x, r = u, u % 2

                def retire():
                    _wait_bytes(rowb[r], yacc.at[didx[x]], ssem[r])
                    if with_g:
                        _wait_bytes(nb[r], ghist.at[sidx[x]], qsem[r])

                if u >= 2:
                    retire()
                else:
                    pl.when(o > 0)(retire)

                _wait_bytes(src_ref.at[pl.ds(0, CH)], sidx[x], isem[x])
                _wait_bytes(dst_ref.at[pl.ds(0, CH)], didx[x], isem[x])
                pltpu.async_copy(tab_ref.at[sidx[x]], rowb[r], gsem[r])
                if with_g:
                    pltpu.async_copy(inorm8_ref.at[didx[x]], nb[r], nsem[r])

                x4 = (u + 4) % UNR
                base4 = s * EC + (o * UNR + u + 4) * CH

                def prefetch():
                    _issue_idx(src_ref, sidx[x4], isem[x4], base4)
                    _issue_idx(dst_ref, didx[x4], isem[x4], base4)

                if u < 4:
                    prefetch()
                else:
                    pl.when(o < NI - 1)(prefetch)

                x1, r1 = (u - 1) % UNR, (u - 1) % 2

                def scatter1():
                    _wait_bytes(tab_ref.at[sidx[x1]], rowb[r1], gsem[r1])
                    pltpu.async_copy(rowb[r1], yacc.at[didx[x1]], ssem[r1], add=True)
                    if with_g:
                        _wait_bytes(inorm8_ref.at[didx[x1]], nb[r1], nsem[r1])
                        pltpu.async_copy(nb[r1], ghist.at[sidx[x1]], qsem[r1], add=True)

                if u >= 1:
                    scatter1()
                else:
                    pl.when(o > 0)(scatter1)

        ctail = NCH - 1
        x1, r1 = ctail % UNR, ctail % 2
        _wait_bytes(tab_ref.at[sidx[x1]], rowb[r1], gsem[r1])
        pltpu.async_copy(rowb[r1], yacc.at[didx[x1]], ssem[r1], add=True)
        if with_g:
            _wait_bytes(inorm8_ref.at[didx[x1]], nb[r1], nsem[r1])
            pltpu.async_copy(nb[r1], ghist.at[sidx[x1]], qsem[r1], add=True)

        for k in range(2):
            _wait_bytes(rowb[k], yacc.at[didx[k]], ssem[k])
            if with_g:
                _wait_bytes(nb[k], ghist.at[sidx[k]], qsem[k])

        plsc.subcore_barrier()
        pltpu.sync_copy(yacc.at[pl.ds(base_r, RS)], out_ref.at[pl.ds(base_r, RS)])
        if with_g:
            pltpu.sync_copy(ghist.at[pl.ds(base_r, RS)], g_ref.at[pl.ds(base_r, RS)])

    @pl.when(c == 0)
    def _():
        agg_pipeline(xa_ref, yp_ref, True)

    @pl.when(c == 1)
    def _():
        agg_pipeline(xb_ref, yn_ref, False)


@functools.partial(
    pl.kernel,
    out_type=(
        jax.ShapeDtypeStruct((NP, SW), jnp.float32),
        jax.ShapeDtypeStruct((NP, SW), jnp.float32),
    ),
    mesh=_mesh,
    compiler_params=_sc_params,
    scratch_types=(
        [pltpu.VMEM((CH,), jnp.int32)] * (2 * UNR) +
        [pltpu.VMEM((CH, SW), jnp.float32)] * 4 +
        [pltpu.VMEM_SHARED((NP, SW), jnp.float32)] +
        [pltpu.SemaphoreType.DMA] * (UNR + 8)
    ),
)
def _k6(src_ref, dst_ref, tp_ref, tn_ref, z8_ref,
        qp_ref, qn_ref, *scr):
    sidx = scr[0:UNR]
    didx = scr[UNR:2 * UNR]
    vb = scr[2 * UNR:2 * UNR + 4]
    qacc = scr[2 * UNR + 4]
    sems = scr[2 * UNR + 5:]
    isem = sems[0:UNR]
    gsem = sems[UNR:UNR + 4]
    ssem = sems[UNR + 4:UNR + 8]

    c = lax.axis_index("c")
    s = lax.axis_index("s")
    base_r = s * RS

    def seg_pipeline(tab_ref, out_ref):
        pltpu.sync_copy(z8_ref.at[pl.ds(base_r, RS)], qacc.at[pl.ds(base_r, RS)])
        plsc.subcore_barrier()

        for k in range(4):
            _issue_idx(src_ref, sidx[k], isem[k], s * EC + k * CH)
            _issue_idx(dst_ref, didx[k], isem[k], s * EC + k * CH)

        @pl.loop(0, NI)
        def _(o):
            for u in range(UNR):
                x, r = u, u % 4

                def retire():
                    _wait_bytes(vb[r], qacc.at[didx[x]], ssem[r])

                if u >= 4:
                    retire()
                else:
                    pl.when(o > 0)(retire)

                _wait_bytes(src_ref.at[pl.ds(0, CH)], sidx[x], isem[x])
                _wait_bytes(dst_ref.at[pl.ds(0, CH)], didx[x], isem[x])
                pltpu.async_copy(tab_ref.at[sidx[x]], vb[r], gsem[r])

                x4 = (u + 4) % UNR
                base4 = s * EC + (o * UNR + u + 4) * CH

                def prefetch():
                    _issue_idx(src_ref, sidx[x4], isem[x4], base4)
                    _issue_idx(dst_ref, didx[x4], isem[x4], base4)

                if u < 4:
                    prefetch()
                else:
                    pl.when(o < NI - 1)(prefetch)

                x2, r2 = (u - 2) % UNR, (u - 2) % 4

                def scatter2():
                    _wait_bytes(tab_ref.at[sidx[x2]], vb[r2], gsem[r2])
                    pltpu.async_copy(vb[r2], qacc.at[didx[x2]], ssem[r2], add=True)

                if u >= 2:
                    scatter2()
                else:
                    pl.when(o > 0)(scatter2)

        for ctail in (NCH - 2, NCH - 1):
            x2, r2 = ctail % UNR, ctail % 4
            _wait_bytes(tab_ref.at[sidx[x2]], vb[r2], gsem[r2])
            pltpu.async_copy(vb[r2], qacc.at[didx[x2]], ssem[r2], add=True)

        for k in range(4):
            _wait_bytes(vb[k], qacc.at[didx[k]], ssem[k])

        plsc.subcore_barrier()
        pltpu.sync_copy(qacc.at[pl.ds(base_r, RS)], out_ref.at[pl.ds(base_r, RS)])

    @pl.when(c == 0)
    def _():
        seg_pipeline(tp_ref, qp_ref)

    @pl.when(c == 1)
    def _():
        seg_pipeline(tn_ref, qn_ref)


def _k2_body(dego_ref, degi_ref, x_ref, xpg_ref,
             onorm_ref, inorm_ref, inorm8_ref, xa_ref, xb_ref):
    on = lax.rsqrt(jnp.maximum(_col0(dego_ref[...]), 1.0))
    inn = lax.rsqrt(jnp.maximum(_col0(degi_ref[...]), 1.0))
    onorm_ref[...] = on
    inorm_ref[...] = inn
    inorm8_ref[...] = inn * _onehot_row(SW)
    xa_ref[...] = (x_ref[...] * on).astype(jnp.bfloat16)
    xb_ref[...] = (xpg_ref[...] * on).astype(jnp.bfloat16)


def _k2(dego, degi, x, xpg):
    return pl.pallas_call(
        _k2_body,
        grid=(GRID,),
        in_specs=[
            pl.BlockSpec((TB, SW), lambda i: (i, 0)),
            pl.BlockSpec((TB, SW), lambda i: (i, 0)),
            pl.BlockSpec((TB, D_IN), lambda i: (i, 0)),
            pl.BlockSpec((TB, D_IN), lambda i: (i, 0)),
        ],
        out_specs=[
            pl.BlockSpec((TB, 1), lambda i: (i, 0)),
            pl.BlockSpec((TB, 1), lambda i: (i, 0)),
            pl.BlockSpec((TB, SW), lambda i: (i, 0)),
            pl.BlockSpec((TB, D_IN), lambda i: (i, 0)),
            pl.BlockSpec((TB, D_IN), lambda i: (i, 0)),
        ],
        out_shape=[
            jax.ShapeDtypeStruct((NP, 1), jnp.float32),
            jax.ShapeDtypeStruct((NP, 1), jnp.float32),
            jax.ShapeDtypeStruct((NP, SW), jnp.float32),
            jax.ShapeDtypeStruct((NP, D_IN), jnp.bfloat16),
            jax.ShapeDtypeStruct((NP, D_IN), jnp.bfloat16),
        ],
    )(dego, degi, x, xpg)


def _k4a_body(yp_ref, yn_ref, inorm_ref, onorm_ref, g_ref, w1_ref, b1_ref,
              a1_ref, h1p_ref, h1n_ref, r_ref):
    inn = inorm_ref[...]
    w1 = w1_ref[...]
    b1 = b1_ref[...]
    a1 = a1_ref[...]
    zp = jnp.dot(yp_ref[...].astype(jnp.float32) * inn, w1,
                 preferred_element_type=jnp.float32) + b1
    hp = jnp.where(zp >= 0.0, zp, a1 * zp)
    h1p_ref[...] = hp
    zn = jnp.dot(yn_ref[...].astype(jnp.float32) * inn, w1,
                 preferred_element_type=jnp.float32) + b1
    h1n_ref[...] = jnp.where(zn >= 0.0, zn, a1 * zn)

    i = pl.program_id(0)
    rowid = i * TB + lax.broadcasted_iota(jnp.int32, (TB, 1), 0)
    gcol = jnp.where(rowid < N, onorm_ref[...] * _col0(g_ref[...]), 0.0)

    @pl.when(i == 0)
    def _():
        r_ref[...] = jnp.zeros_like(r_ref)

    r_ref[...] += jnp.sum(gcol * hp, axis=0, keepdims=True)


def _k4a(yp, yn, inorm, onorm, g, w1, b1, a1):
    return pl.pallas_call(
        _k4a_body,
        grid=(GRID,),
        in_specs=[
            pl.BlockSpec((TB, D_IN), lambda i: (i, 0)),
            pl.BlockSpec((TB, D_IN), lambda i: (i, 0)),
            pl.BlockSpec((TB, 1), lambda i: (i, 0)),
            pl.BlockSpec((TB, 1), lambda i: (i, 0)),
            pl.BlockSpec((TB, SW), lambda i: (i, 0)),
            pl.BlockSpec((D_IN, D_H), lambda i: (0, 0)),
            pl.BlockSpec((1, D_H), lambda i: (0, 0)),
            pl.BlockSpec((1, D_H), lambda i: (0, 0)),
        ],
        out_specs=[
            pl.BlockSpec((TB, D_H), lambda i: (i, 0)),
            pl.BlockSpec((TB, D_H), lambda i: (i, 0)),
            pl.BlockSpec((1, D_H), lambda i: (0, 0)),
        ],
        out_shape=[
            jax.ShapeDtypeStruct((NP, D_H), jnp.float32),
            jax.ShapeDtypeStruct((NP, D_H), jnp.float32),
            jax.ShapeDtypeStruct((1, D_H), jnp.float32),
        ],
    )(yp, yn, inorm, onorm, g, w1, b1, a1)


def _k4b_body(r_ref, w2_ref, w2t_ref, wdt_ref, b2_ref, wt_ref, beta_ref):
    m = jnp.dot(r_ref[...], w2_ref[...], preferred_element_type=jnp.float32) / N
    m = m + b2_ref[...]
    sgm = 1.0 / (1.0 + jnp.exp(-m))
    ws = jnp.dot(sgm, wdt_ref[...], preferred_element_type=jnp.float32)
    wt_ref[...] = jnp.dot(ws, w2t_ref[...], preferred_element_type=jnp.float32)
    beta_ref[...] = jnp.sum(b2_ref[...] * ws, axis=1, keepdims=True)


def _k4b(r, w2, w2t, wdt, b2):
    return pl.pallas_call(
        _k4b_body,
        out_shape=[
            jax.ShapeDtypeStruct((1, D_H), jnp.float32),
            jax.ShapeDtypeStruct((1, 1), jnp.float32),
        ],
    )(r, w2, w2t, wdt, b2)


def _k4c_body(h1p_ref, h1n_ref, onorm_ref, wt_ref, tp_ref, tn_ref):
    wt = wt_ref[...]
    on = onorm_ref[...]
    oh = _onehot_row(SW)
    tp_ref[...] = on * jnp.sum(h1p_ref[...] * wt, axis=1, keepdims=True) * oh
    tn_ref[...] = on * jnp.sum(h1n_ref[...] * wt, axis=1, keepdims=True) * oh


def _k4c(h1p, h1n, onorm, wt):
    return pl.pallas_call(
        _k4c_body,
        grid=(GRID,),
        in_specs=[
            pl.BlockSpec((TB, D_H), lambda i: (i, 0)),
            pl.BlockSpec((TB, D_H), lambda i: (i, 0)),
            pl.BlockSpec((TB, 1), lambda i: (i, 0)),
            pl.BlockSpec((1, D_H), lambda i: (0, 0)),
        ],
        out_specs=[
            pl.BlockSpec((TB, SW), lambda i: (i, 0)),
            pl.BlockSpec((TB, SW), lambda i: (i, 0)),
        ],
        out_shape=[
            jax.ShapeDtypeStruct((NP, SW), jnp.float32),
            jax.ShapeDtypeStruct((NP, SW), jnp.float32),
        ],
    )(h1p, h1n, onorm, wt)


def _k5_body(qp_ref, qn_ref, inorm_ref, beta_ref, loss_ref):
    rows = NP // 128
    rowid = lax.broadcasted_iota(jnp.int32, (rows, 128), 0) * 128 + \
        lax.broadcasted_iota(jnp.int32, (rows, 128), 1)
    mask = rowid < N
    inn = inorm_ref[...]
    beta = beta_ref[...]

    sp = inn * qp_ref[...] + beta
    l1 = jnp.maximum(sp, 0.0) - sp + jnp.log(1.0 + jnp.exp(-jnp.abs(sp)))
    l1 = jnp.where(mask, l1, 0.0)

    sn = inn * qn_ref[...] + beta
    l2 = jnp.maximum(sn, 0.0) + jnp.log(1.0 + jnp.exp(-jnp.abs(sn)))
    l2 = jnp.where(mask, l2, 0.0)

    total = jnp.sum(l1 + l2, axis=0, keepdims=True)
    loss_ref[...] = jnp.sum(total, axis=1, keepdims=True) / N


def _k5(qp2, qn2, inorm2, beta):
    return pl.pallas_call(
        _k5_body,
        out_shape=jax.ShapeDtypeStruct((1, 1), jnp.float32),
    )(qp2, qn2, inorm2, beta)


def kernel(feats, edge_index, W1, b1, a1, W2, b2, Wd):
    src = edge_index[0].astype(jnp.int32)
    dst = edge_index[1].astype(jnp.int32)
    pad_e = jnp.full((EP - E,), DUMMY, dtype=jnp.int32)
    src_p = jnp.concatenate([src, pad_e])
    dst_p = jnp.concatenate([dst, pad_e])

    x_p = jnp.pad(feats.astype(jnp.float32), ((0, NP - N), (0, 0)))
    perm = jax.random.permutation(jax.random.key(1), N).astype(jnp.int32)
    perm_p = jnp.concatenate([perm, jnp.full((NP - N,), DUMMY, dtype=jnp.int32)])

    ones8 = jnp.zeros((CH, SW), jnp.float32).at[:, 0].set(1.0)
    z8 = jnp.zeros((NP, SW), jnp.float32)
    z128 = jnp.zeros((NP, D_IN), jnp.bfloat16)

    dego, degi, xpg = _k1(src_p, dst_p, x_p, perm_p, ones8, z8)
    onorm, inorm, inorm8, xa, xb = _k2(dego, degi, x_p, xpg)
    yp, yn, g = _k3(src_p, dst_p, xa, xb, inorm8, z128, z8)

    w1 = W1.astype(jnp.float32)
    b1r = b1.astype(jnp.float32).reshape(1, D_H)
    a1r = a1.astype(jnp.float32).reshape(1, D_H)
    h1p, h1n, r = _k4a(yp, yn, inorm, onorm, g, w1, b1r, a1r)

    w2 = W2.astype(jnp.float32)
    b2r = b2.astype(jnp.float32).reshape(1, D_H)
    wt, beta = _k4b(r, w2, w2.T, Wd.astype(jnp.float32).T, b2r)

    tp, tn = _k4c(h1p, h1n, onorm, wt)
    qp, qn = _k6(src_p, dst_p, tp, tn, z8)

    loss = _k5(qp[:, 0].reshape(NP // 128, 128), qn[:, 0].reshape(NP // 128, 128),
               inorm.reshape(NP // 128, 128), beta)
    return loss[0, 0]


# speedup vs baseline: 15.4739x; 1.0341x over previous
import functools

import jax
import jax.numpy as jnp
from jax import lax
from jax.experimental import pallas as pl
from jax.experimental.pallas import tpu as pltpu
from jax.experimental.pallas import tpu_sc as plsc

N = 10000
E = 320000
D_IN = 128
D_H = 512

NP = 10240
DUMMY = N
NSUB = 16
RS = NP // NSUB
CH = 128
NCH = 160
EC = NCH * CH
EP = NSUB * EC
UNR = 8
NI = NCH // UNR
SW = 8
TB = 256
GRID = NP // TB

_mesh = plsc.VectorSubcoreMesh(core_axis_name="c", subcore_axis_name="s")
_sc_params = pltpu.CompilerParams(use_tc_tiling_on_sc=False)


def _col0(block):
    return block[:, 0:1]


def _issue_idx(arr_ref, buf, sem, base):
    pltpu.async_copy(arr_ref.at[pl.ds(base, CH)], buf, sem)


def _wait_bytes(src, dst, sem):
    pltpu.make_async_copy(src, dst, sem).wait()


def _onehot_row(width):
    return (lax.broadcasted_iota(jnp.int32, (1, width), 1) == 0).astype(jnp.float32)


@functools.partial(
    pl.kernel,
    out_type=(
        jax.ShapeDtypeStruct((NP, SW), jnp.float32),
        jax.ShapeDtypeStruct((NP, SW), jnp.float32),
        jax.ShapeDtypeStruct((NP, D_IN), jnp.float32),
    ),
    mesh=_mesh,
    compiler_params=_sc_params,
    scratch_types=(
        [pltpu.VMEM((CH,), jnp.int32)] * UNR +
        [pltpu.VMEM((CH,), jnp.int32)] +
        [pltpu.VMEM((CH, SW), jnp.float32)] +
        [pltpu.VMEM((CH, D_IN), jnp.float32)] +
        [pltpu.VMEM_SHARED((NP, SW), jnp.float32)] +
        [pltpu.SemaphoreType.DMA] * UNR +
        [pltpu.SemaphoreType.DMA] * 4
    ),
)
def _k1(src_ref, dst_ref, x_ref, perm_ref, ones_ref, z8_ref,
        dego_ref, degi_ref, xpg_ref, *scr):
    idxb = scr[0:UNR]
    pidx = scr[UNR]
    onesb = scr[UNR + 1]
    rowb = scr[UNR + 2]
    hist = scr[UNR + 3]
    isem = scr[UNR + 4:UNR + 4 + UNR]
    ssem = scr[UNR + 4 + UNR:UNR + 8 + UNR]

    c = lax.axis_index("c")
    s = lax.axis_index("s")
    base_r = s * RS

    def hist_pipeline(earr, out_ref):
        pltpu.sync_copy(ones_ref, onesb)
        pltpu.sync_copy(z8_ref.at[pl.ds(base_r, RS)], hist.at[pl.ds(base_r, RS)])
        plsc.subcore_barrier()

        for k in range(4):
            _issue_idx(earr, idxb[k], isem[k], s * EC + k * CH)

        @pl.loop(0, NI)
        def _(o):
            for u in range(UNR):
                x, r = u, u % 4

                def waits():
                    _wait_bytes(onesb, hist.at[idxb[x]], ssem[r])

                if u >= 4:
                    waits()
                else:
                    pl.when(o > 0)(waits)

                _wait_bytes(earr.at[pl.ds(0, CH)], idxb[x], isem[x])
                pltpu.async_copy(onesb, hist.at[idxb[x]], ssem[r], add=True)

                x4 = (u + 4) % UNR
                base4 = s * EC + (o * UNR + u + 4) * CH

                def prefetch():
                    _issue_idx(earr, idxb[x4], isem[x4], base4)

                if u < 4:
                    prefetch()
                else:
                    pl.when(o < NI - 1)(prefetch)

        for k in range(4):
            _wait_bytes(onesb, hist.at[idxb[k]], ssem[k])

        plsc.subcore_barrier()
        pltpu.sync_copy(hist.at[pl.ds(base_r, RS)], out_ref.at[pl.ds(base_r, RS)])

    @pl.when(c == 0)
    def _():
        @pl.loop(0, RS // CH)
        def _(j):
            base = base_r + j * CH
            pltpu.sync_copy(perm_ref.at[pl.ds(base, CH)], pidx)
            pltpu.sync_copy(x_ref.at[pidx], rowb)
            pltpu.sync_copy(rowb, xpg_ref.at[pl.ds(base, CH)])

        hist_pipeline(src_ref, dego_ref)

    @pl.when(c == 1)
    def _():
        hist_pipeline(dst_ref, degi_ref)


@functools.partial(
    pl.kernel,
    out_type=(
        jax.ShapeDtypeStruct((NP, D_IN), jnp.bfloat16),
        jax.ShapeDtypeStruct((NP, D_IN), jnp.bfloat16),
        jax.ShapeDtypeStruct((NP, SW), jnp.float32),
    ),
    mesh=_mesh,
    compiler_params=_sc_params,
    scratch_types=(
        [pltpu.VMEM((CH,), jnp.int32)] * (2 * UNR) +
        [pltpu.VMEM((CH, D_IN), jnp.bfloat16)] * 2 +
        [pltpu.VMEM((CH, SW), jnp.float32)] * 2 +
        [pltpu.VMEM_SHARED((NP, D_IN), jnp.bfloat16)] +
        [pltpu.VMEM_SHARED((NP, SW), jnp.float32)] +
        [pltpu.SemaphoreType.DMA] * (UNR + 8)
    ),
)
def _k3(src_ref, dst_ref, xa_ref, xb_ref, inorm8_ref, z128_ref, z8_ref,
        yp_ref, yn_ref, g_ref, *scr):
    sidx = scr[0:UNR]
    didx = scr[UNR:2 * UNR]
    rowb = scr[2 * UNR:2 * UNR + 2]
    nb = scr[2 * UNR + 2:2 * UNR + 4]
    yacc = scr[2 * UNR + 4]
    ghist = scr[2 * UNR + 5]
    sems = scr[2 * UNR + 6:]
    isem = sems[0:UNR]
    gsem = sems[UNR:UNR + 2]
    ssem = sems[UNR + 2:UNR + 4]
    nsem = sems[UNR + 4:UNR + 6]
    qsem = sems[UNR + 6:UNR + 8]

    c = lax.axis_index("c")
    s = lax.axis_index("s")
    base_r = s * RS

    def agg_pipeline(tab_ref, out_ref, with_g):
        pltpu.sync_copy(z128_ref.at[pl.ds(base_r, RS)], yacc.at[pl.ds(base_r, RS)])
        if with_g:
            pltpu.sync_copy(z8_ref.at[pl.ds(base_r, RS)], ghist.at[pl.ds(base_r, RS)])
        plsc.subcore_barrier()

        for k in range(4):
            _issue_idx(src_ref, sidx[k], isem[k], s * EC + k * CH)
            _issue_idx(dst_ref, didx[k], isem[k], s * EC + k * CH)

        @pl.loop(0, NI)
        def _(o):
            for u in range(UNR):
                x, r = u, u % 2

                def retire():
                    _wait_bytes(rowb[r], yacc.at[didx[x]], ssem[r])
                    if with_g:
                        _wait_bytes(nb[r], ghist.at[sidx[x]], qsem[r])

                if u >= 2:
                    retire()
                else:
                    pl.when(o > 0)(retire)

                _wait_bytes(src_ref.at[pl.ds(0, CH)], sidx[x], isem[x])
                _wait_bytes(dst_ref.at[pl.ds(0, CH)], didx[x], isem[x])
                pltpu.async_copy(tab_ref.at[sidx[x]], rowb[r], gsem[r])
                if with_g:
                    pltpu.async_copy(inorm8_ref.at[didx[x]], nb[r], nsem[r])

                x4 = (u + 4) % UNR
                base4 = s * EC + (o * UNR + u + 4) * CH

                def prefetch():
                    _issue_idx(src_ref, sidx[x4], isem[x4], base4)
                    _issue_idx(dst_ref, didx[x4], isem[x4], base4)

                if u < 4:
                    prefetch()
                else:
                    pl.when(o < NI - 1)(prefetch)

                x1, r1 = (u - 1) % UNR, (u - 1) % 2

                def scatter1():
                    _wait_bytes(tab_ref.at[sidx[x1]], rowb[r1], gsem[r1])
                    pltpu.async_copy(rowb[r1], yacc.at[didx[x1]], ssem[r1], add=True)
                    if with_g:
                        _wait_bytes(inorm8_ref.at[didx[x1]], nb[r1], nsem[r1])
                        pltpu.async_copy(nb[r1], ghist.at[sidx[x1]], qsem[r1], add=True)

                if u >= 1:
                    scatter1()
                else:
                    pl.when(o > 0)(scatter1)

        ctail = NCH - 1
        x1, r1 = ctail % UNR, ctail % 2
        _wait_bytes(tab_ref.at[sidx[x1]], rowb[r1], gsem[r1])
        pltpu.async_copy(rowb[r1], yacc.at[didx[x1]], ssem[r1], add=True)
        if with_g:
            _wait_bytes(inorm8_ref.at[didx[x1]], nb[r1], nsem[r1])
            pltpu.async_copy(nb[r1], ghist.at[sidx[x1]], qsem[r1], add=True)

        for k in range(2):
            _wait_bytes(rowb[k], yacc.at[didx[k]], ssem[k])
            if with_g:
                _wait_bytes(nb[k], ghist.at[sidx[k]], qsem[k])

        plsc.subcore_barrier()
        pltpu.sync_copy(yacc.at[pl.ds(base_r, RS)], out_ref.at[pl.ds(base_r, RS)])
        if with_g:
            pltpu.sync_copy(ghist.at[pl.ds(base_r, RS)], g_ref.at[pl.ds(base_r, RS)])

    @pl.when(c == 0)
    def _():
        agg_pipeline(xa_ref, yp_ref, True)

    @pl.when(c == 1)
    def _():
        agg_pipeline(xb_ref, yn_ref, False)


@functools.partial(
    pl.kernel,
    out_type=(
        jax.ShapeDtypeStruct((NP, SW), jnp.float32),
        jax.ShapeDtypeStruct((NP, SW), jnp.float32),
    ),
    mesh=_mesh,
    compiler_params=_sc_params,
    scratch_types=(
        [pltpu.VMEM((CH,), jnp.int32)] * (2 * UNR) +
        [pltpu.VMEM((CH, SW), jnp.float32)] * 4 +
        [pltpu.VMEM_SHARED((NP, SW), jnp.float32)] +
        [pltpu.SemaphoreType.DMA] * (UNR + 8)
    ),
)
def _k6(src_ref, dst_ref, tp_ref, tn_ref, z8_ref,
        qp_ref, qn_ref, *scr):
    sidx = scr[0:UNR]
    didx = scr[UNR:2 * UNR]
    vb = scr[2 * UNR:2 * UNR + 4]
    qacc = scr[2 * UNR + 4]
    sems = scr[2 * UNR + 5:]
    isem = sems[0:UNR]
    gsem = sems[UNR:UNR + 4]
    ssem = sems[UNR + 4:UNR + 8]

    c = lax.axis_index("c")
    s = lax.axis_index("s")
    base_r = s * RS

    def seg_pipeline(tab_ref, out_ref):
        pltpu.sync_copy(z8_ref.at[pl.ds(base_r, RS)], qacc.at[pl.ds(base_r, RS)])
        plsc.subcore_barrier()

        for k in range(4):
            _issue_idx(src_ref, sidx[k], isem[k], s * EC + k * CH)
            _issue_idx(dst_ref, didx[k], isem[k], s * EC + k * CH)

        @pl.loop(0, NI)
        def _(o):
            for u in range(UNR):
                x, r = u, u % 4

                def retire():
                    _wait_bytes(vb[r], qacc.at[didx[x]], ssem[r])

                if u >= 4:
                    retire()
                else:
                    pl.when(o > 0)(retire)

                _wait_bytes(src_ref.at[pl.ds(0, CH)], sidx[x], isem[x])
                _wait_bytes(dst_ref.at[pl.ds(0, CH)], didx[x], isem[x])
                pltpu.async_copy(tab_ref.at[sidx[x]], vb[r], gsem[r])

                x4 = (u + 4) % UNR
                base4 = s * EC + (o * UNR + u + 4) * CH

                def prefetch():
                    _issue_idx(src_ref, sidx[x4], isem[x4], base4)
                    _issue_idx(dst_ref, didx[x4], isem[x4], base4)

                if u < 4:
                    prefetch()
                else:
                    pl.when(o < NI - 1)(prefetch)

                x2, r2 = (u - 2) % UNR, (u - 2) % 4

                def scatter2():
                    _wait_bytes(tab_ref.at[sidx[x2]], vb[r2], gsem[r2])
                    pltpu.async_copy(vb[r2], qacc.at[didx[x2]], ssem[r2], add=True)

                if u >= 2:
                    scatter2()
                else:
                    pl.when(o > 0)(scatter2)

        for ctail in (NCH - 2, NCH - 1):
            x2, r2 = ctail % UNR, ctail % 4
            _wait_bytes(tab_ref.at[sidx[x2]], vb[r2], gsem[r2])
            pltpu.async_copy(vb[r2], qacc.at[didx[x2]], ssem[r2], add=True)

        for k in range(4):
            _wait_bytes(vb[k], qacc.at[didx[k]], ssem[k])

        plsc.subcore_barrier()
        pltpu.sync_copy(qacc.at[pl.ds(base_r, RS)], out_ref.at[pl.ds(base_r, RS)])

    @pl.when(c == 0)
    def _():
        seg_pipeline(tp_ref, qp_ref)

    @pl.when(c == 1)
    def _():
        seg_pipeline(tn_ref, qn_ref)


def _k2_body(dego_ref, degi_ref, x_ref, xpg_ref,
             onorm_ref, inorm_ref, inorm8_ref, xa_ref, xb_ref):
    on = lax.rsqrt(jnp.maximum(_col0(dego_ref[...]), 1.0))
    inn = lax.rsqrt(jnp.maximum(_col0(degi_ref[...]), 1.0))
    onorm_ref[...] = on
    inorm_ref[...] = inn
    inorm8_ref[...] = inn * _onehot_row(SW)
    xa_ref[...] = (x_ref[...] * on).astype(jnp.bfloat16)
    xb_ref[...] = (xpg_ref[...] * on).astype(jnp.bfloat16)


def _k2(dego, degi, x, xpg):
    return pl.pallas_call(
        _k2_body,
        grid=(GRID,),
        in_specs=[
            pl.BlockSpec((TB, SW), lambda i: (i, 0)),
            pl.BlockSpec((TB, SW), lambda i: (i, 0)),
            pl.BlockSpec((TB, D_IN), lambda i: (i, 0)),
            pl.BlockSpec((TB, D_IN), lambda i: (i, 0)),
        ],
        out_specs=[
            pl.BlockSpec((TB, 1), lambda i: (i, 0)),
            pl.BlockSpec((TB, 1), lambda i: (i, 0)),
            pl.BlockSpec((TB, SW), lambda i: (i, 0)),
            pl.BlockSpec((TB, D_IN), lambda i: (i, 0)),
            pl.BlockSpec((TB, D_IN), lambda i: (i, 0)),
        ],
        out_shape=[
            jax.ShapeDtypeStruct((NP, 1), jnp.float32),
            jax.ShapeDtypeStruct((NP, 1), jnp.float32),
            jax.ShapeDtypeStruct((NP, SW), jnp.float32),
            jax.ShapeDtypeStruct((NP, D_IN), jnp.bfloat16),
            jax.ShapeDtypeStruct((NP, D_IN), jnp.bfloat16),
        ],
    )(dego, degi, x, xpg)


def _prelu_tile(y_ref, inn, w1, b1, a1):
    z = jnp.dot(y_ref[...].astype(jnp.float32) * inn, w1,
                preferred_element_type=jnp.float32) + b1
    return jnp.where(z >= 0.0, z, a1 * z)


def _k4a_body(yp_ref, inorm_ref, onorm_ref, g_ref, w1_ref, b1_ref,
              a1_ref, r_ref):
    inn = inorm_ref[...]
    hp = _prelu_tile(yp_ref, inn, w1_ref[...], b1_ref[...], a1_ref[...])

    i = pl.program_id(0)
    rowid = i * TB + lax.broadcasted_iota(jnp.int32, (TB, 1), 0)
    gcol = jnp.where(rowid < N, onorm_ref[...] * _col0(g_ref[...]), 0.0)

    @pl.when(i == 0)
    def _():
        r_ref[...] = jnp.zeros_like(r_ref)

    r_ref[...] += jnp.sum(gcol * hp, axis=0, keepdims=True)


def _k4a(yp, inorm, onorm, g, w1, b1, a1):
    return pl.pallas_call(
        _k4a_body,
        grid=(GRID,),
        in_specs=[
            pl.BlockSpec((TB, D_IN), lambda i: (i, 0)),
            pl.BlockSpec((TB, 1), lambda i: (i, 0)),
            pl.BlockSpec((TB, 1), lambda i: (i, 0)),
            pl.BlockSpec((TB, SW), lambda i: (i, 0)),
            pl.BlockSpec((D_IN, D_H), lambda i: (0, 0)),
            pl.BlockSpec((1, D_H), lambda i: (0, 0)),
            pl.BlockSpec((1, D_H), lambda i: (0, 0)),
        ],
        out_specs=[
            pl.BlockSpec((1, D_H), lambda i: (0, 0)),
        ],
        out_shape=[
            jax.ShapeDtypeStruct((1, D_H), jnp.float32),
        ],
    )(yp, inorm, onorm, g, w1, b1, a1)


def _k4c_body(yp_ref, yn_ref, inorm_ref, onorm_ref, r_ref, w2_ref, w2t_ref,
              wdt_ref, b2_ref, w1_ref, b1_ref, a1_ref,
              tp_ref, tn_ref, beta_ref, wt_sc):
    @pl.when(pl.program_id(0) == 0)
    def _():
        m = jnp.dot(r_ref[...], w2_ref[...], preferred_element_type=jnp.float32) / N
        m = m + b2_ref[...]
        sgm = 1.0 / (1.0 + jnp.exp(-m))
        ws = jnp.dot(sgm, wdt_ref[...], preferred_element_type=jnp.float32)
        wt_sc[...] = jnp.dot(ws, w2t_ref[...], preferred_element_type=jnp.float32)
        beta_ref[...] = jnp.sum(b2_ref[...] * ws, axis=1, keepdims=True)

    inn = inorm_ref[...]
    w1 = w1_ref[...]
    b1 = b1_ref[...]
    a1 = a1_ref[...]
    wt = wt_sc[...]
    on = onorm_ref[...]
    oh = _onehot_row(SW)
    hp = _prelu_tile(yp_ref, inn, w1, b1, a1)
    tp_ref[...] = on * jnp.sum(hp * wt, axis=1, keepdims=True) * oh
    hn = _prelu_tile(yn_ref, inn, w1, b1, a1)
    tn_ref[...] = on * jnp.sum(hn * wt, axis=1, keepdims=True) * oh


def _k4c(yp, yn, inorm, onorm, r, w2, w2t, wdt, b2, w1, b1, a1):
    return pl.pallas_call(
        _k4c_body,
        grid=(GRID,),
        in_specs=[
            pl.BlockSpec((TB, D_IN), lambda i: (i, 0)),
            pl.BlockSpec((TB, D_IN), lambda i: (i, 0)),
            pl.BlockSpec((TB, 1), lambda i: (i, 0)),
            pl.BlockSpec((TB, 1), lambda i: (i, 0)),
            pl.BlockSpec((1, D_H), lambda i: (0, 0)),
            pl.BlockSpec((D_H, D_H), lambda i: (0, 0)),
            pl.BlockSpec((D_H, D_H), lambda i: (0, 0)),
            pl.BlockSpec((D_H, D_H), lambda i: (0, 0)),
            pl.BlockSpec((1, D_H), lambda i: (0, 0)),
            pl.BlockSpec((D_IN, D_H), lambda i: (0, 0)),
            pl.BlockSpec((1, D_H), lambda i: (0, 0)),
            pl.BlockSpec((1, D_H), lambda i: (0, 0)),
        ],
        out_specs=[
            pl.BlockSpec((TB, SW), lambda i: (i, 0)),
            pl.BlockSpec((TB, SW), lambda i: (i, 0)),
            pl.BlockSpec((1, 1), lambda i: (0, 0)),
        ],
        out_shape=[
            jax.ShapeDtypeStruct((NP, SW), jnp.float32),
            jax.ShapeDtypeStruct((NP, SW), jnp.float32),
            jax.ShapeDtypeStruct((1, 1), jnp.float32),
        ],
        scratch_shapes=[pltpu.VMEM((1, D_H), jnp.float32)],
    )(yp, yn, inorm, onorm, r, w2, w2t, wdt, b2, w1, b1, a1)


def _k5_body(qp_ref, qn_ref, inorm_ref, beta_ref, loss_ref):
    rows = NP // 128
    rowid = lax.broadcasted_iota(jnp.int32, (rows, 128), 0) * 128 + \
        lax.broadcasted_iota(jnp.int32, (rows, 128), 1)
    mask = rowid < N
    inn = inorm_ref[...]
    beta = beta_ref[...]

    sp = inn * qp_ref[...] + beta
    l1 = jnp.maximum(sp, 0.0) - sp + jnp.log(1.0 + jnp.exp(-jnp.abs(sp)))
    l1 = jnp.where(mask, l1, 0.0)

    sn = inn * qn_ref[...] + beta
    l2 = jnp.maximum(sn, 0.0) + jnp.log(1.0 + jnp.exp(-jnp.abs(sn)))
    l2 = jnp.where(mask, l2, 0.0)

    total = jnp.sum(l1 + l2, axis=0, keepdims=True)
    loss_ref[...] = jnp.sum(total, axis=1, keepdims=True) / N


def _k5(qp2, qn2, inorm2, beta):
    return pl.pallas_call(
        _k5_body,
        out_shape=jax.ShapeDtypeStruct((1, 1), jnp.float32),
    )(qp2, qn2, inorm2, beta)


def kernel(feats, edge_index, W1, b1, a1, W2, b2, Wd):
    src = edge_index[0].astype(jnp.int32)
    dst = edge_index[1].astype(jnp.int32)
    pad_e = jnp.full((EP - E,), DUMMY, dtype=jnp.int32)
    src_p = jnp.concatenate([src, pad_e])
    dst_p = jnp.concatenate([dst, pad_e])

    x_p = jnp.pad(feats.astype(jnp.float32), ((0, NP - N), (0, 0)))
    perm = jax.random.permutation(jax.random.key(1), N).astype(jnp.int32)
    perm_p = jnp.concatenate([perm, jnp.full((NP - N,), DUMMY, dtype=jnp.int32)])

    ones8 = jnp.zeros((CH, SW), jnp.float32).at[:, 0].set(1.0)
    z8 = jnp.zeros((NP, SW), jnp.float32)
    z128 = jnp.zeros((NP, D_IN), jnp.bfloat16)

    dego, degi, xpg = _k1(src_p, dst_p, x_p, perm_p, ones8, z8)
    onorm, inorm, inorm8, xa, xb = _k2(dego, degi, x_p, xpg)
    yp, yn, g = _k3(src_p, dst_p, xa, xb, inorm8, z128, z8)

    w1 = W1.astype(jnp.float32)
    b1r = b1.astype(jnp.float32).reshape(1, D_H)
    a1r = a1.astype(jnp.float32).reshape(1, D_H)
    (r,) = _k4a(yp, inorm, onorm, g, w1, b1r, a1r)

    w2 = W2.astype(jnp.float32)
    b2r = b2.astype(jnp.float32).reshape(1, D_H)
    tp, tn, beta = _k4c(yp, yn, inorm, onorm, r, w2, w2.T,
                        Wd.astype(jnp.float32).T, b2r, w1, b1r, a1r)
    qp, qn = _k6(src_p, dst_p, tp, tn, z8)

    loss = _k5(qp[:, 0].reshape(NP // 128, 128), qn[:, 0].reshape(NP // 128, 128),
               inorm.reshape(NP // 128, 128), beta)
    return loss[0, 0]
